# Initial kernel scaffold; baseline (speedup 1.0000x reference)
import jax
import jax.numpy as jnp
from jax import lax
from jax.experimental import pallas as pl
from jax.experimental.pallas import tpu as pltpu
from jax.experimental.pallas import tpu_sc as plsc

N_NODES = 10000
N_EDGES = 10000
NNZ = 320000
CH = 128
PAD = 10240
PAD_ROW = PAD - 1
NC = 2
NS = 16
NW = NC * NS
CHUNK = 128
WCHUNKS = 80
WNNZ = WCHUNKS * CHUNK
ACHUNKS = WCHUNKS * 2
SL = PAD // NS
BLK = 256


def _rsqrt_nr(d):
    i = lax.bitcast_convert_type(d, jnp.int32)
    y = lax.bitcast_convert_type(jnp.int32(0x5F3759DF) - (i >> 1), jnp.float32)
    for _ in range(3):
        y = y * (1.5 - 0.5 * d * y * y)
    return y


def _scales_body(vp_ref, ep_ref, sv_ref, se_ref, snv_ref, sn_ref,
                 vb, eb, ones, ta, tb, vv,
                 dv_s, de_s, ne_s, nv_s, wv_s, we_s, sem):
    c = lax.axis_index("c")
    s = lax.axis_index("s")
    base = s * SL
    pltpu.sync_copy(vp_ref.at[s], vb)
    pltpu.sync_copy(ep_ref.at[s], eb)

    def z16(i, _):
        ta[pl.ds(i * 16, 16)] = jnp.zeros((16,), jnp.float32)
        return 0
    lax.fori_loop(0, SL // 16, z16, 0)

    def o16(i, _):
        ones[pl.ds(i * 16, 16)] = jnp.ones((16,), jnp.float32)
        return 0
    lax.fori_loop(0, CHUNK // 16, o16, 0)

    pltpu.sync_copy(ta, dv_s.at[pl.ds(base, SL)])
    pltpu.sync_copy(ta, de_s.at[pl.ds(base, SL)])
    pltpu.sync_copy(ta, ne_s.at[pl.ds(base, SL)])
    pltpu.sync_copy(ta, nv_s.at[pl.ds(base, SL)])
    plsc.subcore_barrier()

    def deg(i, _):
        pltpu.sync_copy(ones, dv_s.at[vb.at[i]], add=True)
        pltpu.sync_copy(ones, de_s.at[eb.at[i]], add=True)
        return 0
    lax.fori_loop(0, ACHUNKS, deg, 0)
    plsc.subcore_barrier()

    pltpu.sync_copy(dv_s.at[pl.ds(base, SL)], ta)
    pltpu.sync_copy(de_s.at[pl.ds(base, SL)], tb)

    def wvl(i, _):
        dv = jnp.maximum(ta[pl.ds(i * 16, 16)], 1.0)
        ta[pl.ds(i * 16, 16)] = _rsqrt_nr(dv)
        de = jnp.maximum(tb[pl.ds(i * 16, 16)], 1.0)
        r = _rsqrt_nr(de)
        tb[pl.ds(i * 16, 16)] = r * r * r
        return 0
    lax.fori_loop(0, SL // 16, wvl, 0)

    pltpu.sync_copy(ta, wv_s.at[pl.ds(base, SL)])
    pltpu.sync_copy(tb, we_s.at[pl.ds(base, SL)])

    @pl.when(c == 0)
    def _():
        pltpu.sync_copy(ta, sv_ref.at[pl.ds(base, SL)])
    plsc.subcore_barrier()

    def nrm(i, _):
        pltpu.async_copy(wv_s.at[vb.at[i]], vv, sem).wait()
        pltpu.sync_copy(vv, ne_s.at[eb.at[i]], add=True)
        pltpu.async_copy(we_s.at[eb.at[i]], vv, sem).wait()
        pltpu.sync_copy(vv, nv_s.at[vb.at[i]], add=True)
        return 0
    lax.fori_loop(0, ACHUNKS, nrm, 0)
    plsc.subcore_barrier()

    pltpu.sync_copy(ne_s.at[pl.ds(base, SL)], ta)

    def sel(i, _):
        ne = jnp.maximum(ta[pl.ds(i * 16, 16)], 1e-6)
        ta[pl.ds(i * 16, 16)] = tb[pl.ds(i * 16, 16)] / ne
        return 0
    lax.fori_loop(0, SL // 16, sel, 0)

    @pl.when(c == 0)
    def _():
        pltpu.sync_copy(ta, se_ref.at[pl.ds(base, SL)])

    pltpu.sync_copy(nv_s.at[pl.ds(base, SL)], ta)
    pltpu.sync_copy(wv_s.at[pl.ds(base, SL)], tb)

    def snl(i, _):
        sn = 1.0 / jnp.maximum(ta[pl.ds(i * 16, 16)], 1e-6)
        ta[pl.ds(i * 16, 16)] = sn
        tb[pl.ds(i * 16, 16)] = sn * tb[pl.ds(i * 16, 16)]
        return 0
    lax.fori_loop(0, SL // 16, snl, 0)

    @pl.when(c == 0)
    def _():
        pltpu.sync_copy(ta, sn_ref.at[pl.ds(base, SL)])
        pltpu.sync_copy(tb, snv_ref.at[pl.ds(base, SL)])


def _scales(vp16, ep16):
    mesh = plsc.VectorSubcoreMesh(core_axis_name="c", subcore_axis_name="s",
                                  num_cores=NC)
    f = pl.kernel(
        _scales_body,
        out_type=[jax.ShapeDtypeStruct((PAD,), jnp.float32)] * 4,
        mesh=mesh,
        scratch_types=[
            pltpu.VMEM((ACHUNKS, CHUNK), jnp.int32),
            pltpu.VMEM((ACHUNKS, CHUNK), jnp.int32),
            pltpu.VMEM((CHUNK,), jnp.float32),
            pltpu.VMEM((SL,), jnp.float32),
            pltpu.VMEM((SL,), jnp.float32),
            pltpu.VMEM((CHUNK,), jnp.float32),
            pltpu.VMEM_SHARED((PAD,), jnp.float32),
            pltpu.VMEM_SHARED((PAD,), jnp.float32),
            pltpu.VMEM_SHARED((PAD,), jnp.float32),
            pltpu.VMEM_SHARED((PAD,), jnp.float32),
            pltpu.VMEM_SHARED((PAD,), jnp.float32),
            pltpu.VMEM_SHARED((PAD,), jnp.float32),
            pltpu.SemaphoreType.DMA,
        ],
    )
    return f(vp16, ep16)


def _spmm_body(table_ref, gi_ref, si_ref, out_ref,
               vb, eb, r0, r1, stage, acc, sem0, sem1):
    c = lax.axis_index("c")
    s = lax.axis_index("s")
    wid = s * NC + c
    pltpu.sync_copy(gi_ref.at[wid], vb)
    pltpu.sync_copy(si_ref.at[wid], eb)

    def zr(rr, _):
        def zc(j, __):
            stage[rr, pl.ds(j * 16, 16)] = jnp.zeros((16,), jnp.float32)
            return 0
        lax.fori_loop(0, CH // 16, zc, 0)
        return 0
    lax.fori_loop(0, CHUNK, zr, 0)

    def za(k, _):
        pltpu.sync_copy(stage, acc.at[pl.ds(s * SL + k * CHUNK, CHUNK)])
        return 0
    lax.fori_loop(0, SL // CHUNK, za, 0)
    plsc.subcore_barrier()

    def mo(i, _):
        d0 = pltpu.async_copy(table_ref.at[vb.at[2 * i]], r0, sem0)
        d1 = pltpu.async_copy(table_ref.at[vb.at[2 * i + 1]], r1, sem1)
        d0.wait()
        pltpu.sync_copy(r0, acc.at[eb.at[2 * i]], add=True)
        d1.wait()
        pltpu.sync_copy(r1, acc.at[eb.at[2 * i + 1]], add=True)
        return 0
    lax.fori_loop(0, WCHUNKS // 2, mo, 0)
    plsc.subcore_barrier()

    def wb(k, _):
        row = s * SL + k * CHUNK
        pltpu.sync_copy(acc.at[pl.ds(row, CHUNK)], stage)
        pltpu.sync_copy(stage, out_ref.at[c, pl.ds(row, CHUNK)])
        return 0
    lax.fori_loop(0, SL // CHUNK, wb, 0)


def _spmm(table, gidx, sidx):
    mesh = plsc.VectorSubcoreMesh(core_axis_name="c", subcore_axis_name="s",
                                  num_cores=NC)
    f = pl.kernel(
        _spmm_body,
        out_type=jax.ShapeDtypeStruct((NC, PAD, CH), jnp.float32),
        mesh=mesh,
        scratch_types=[
            pltpu.VMEM((WCHUNKS, CHUNK), jnp.int32),
            pltpu.VMEM((WCHUNKS, CHUNK), jnp.int32),
            pltpu.VMEM((CHUNK, CH), jnp.float32),
            pltpu.VMEM((CHUNK, CH), jnp.float32),
            pltpu.VMEM((CHUNK, CH), jnp.float32),
            pltpu.VMEM_SHARED((PAD, CH), jnp.float32),
            pltpu.SemaphoreType.DMA,
            pltpu.SemaphoreType.DMA,
        ],
    )
    return f(table, gidx, sidx)


def _tbl_first_body(x_ref, s_ref, w_ref, o_ref):
    o_ref[...] = lax.dot_general(x_ref[...] * s_ref[...], w_ref[...],
                                 (((1,), (0,)), ((), ())),
                                 preferred_element_type=jnp.float32)


def _tbl_mid_body(p0_ref, p1_ref, s_ref, w_ref, o_ref):
    h = jnp.maximum(p0_ref[...] + p1_ref[...], 0.0) * s_ref[...]
    o_ref[...] = lax.dot_general(h, w_ref[...], (((1,), (0,)), ((), ())),
                                 preferred_element_type=jnp.float32)


def _max_body(p0_ref, p1_ref, s_ref, o_ref):
    i = pl.program_id(0)
    h = jnp.maximum(p0_ref[...] + p1_ref[...], 0.0) * s_ref[...]
    m = jnp.max(h, axis=0, keepdims=True)

    @pl.when(i == 0)
    def _():
        o_ref[...] = m

    @pl.when(i != 0)
    def _():
        o_ref[...] = jnp.maximum(o_ref[...], m)


def _dot_body(p_ref, w_ref, b_ref, o_ref):
    o_ref[...] = lax.dot_general(p_ref[...], w_ref[...],
                                 (((1,), (0,)), ((), ())),
                                 preferred_element_type=jnp.float32) + b_ref[...]


def _tbl_first(x, sv, w):
    return pl.pallas_call(
        _tbl_first_body,
        grid=(PAD // BLK,),
        in_specs=[pl.BlockSpec((BLK, CH), lambda i: (i, 0)),
                  pl.BlockSpec((BLK, 1), lambda i: (i, 0)),
                  pl.BlockSpec((CH, CH), lambda i: (0, 0))],
        out_specs=pl.BlockSpec((BLK, CH), lambda i: (i, 0)),
        out_shape=jax.ShapeDtypeStruct((PAD, CH), jnp.float32),
    )(x, sv, w)


def _tbl_mid(p0, p1, sc, w):
    return pl.pallas_call(
        _tbl_mid_body,
        grid=(PAD // BLK,),
        in_specs=[pl.BlockSpec((BLK, CH), lambda i: (i, 0)),
                  pl.BlockSpec((BLK, CH), lambda i: (i, 0)),
                  pl.BlockSpec((BLK, 1), lambda i: (i, 0)),
                  pl.BlockSpec((CH, CH), lambda i: (0, 0))],
        out_specs=pl.BlockSpec((BLK, CH), lambda i: (i, 0)),
        out_shape=jax.ShapeDtypeStruct((PAD, CH), jnp.float32),
    )(p0, p1, sc, w)


def _maxpool(p0, p1, sc):
    return pl.pallas_call(
        _max_body,
        grid=(PAD // BLK,),
        in_specs=[pl.BlockSpec((BLK, CH), lambda i: (i, 0)),
                  pl.BlockSpec((BLK, CH), lambda i: (i, 0)),
                  pl.BlockSpec((BLK, 1), lambda i: (i, 0))],
        out_specs=pl.BlockSpec((1, CH), lambda i: (0, 0)),
        out_shape=jax.ShapeDtypeStruct((1, CH), jnp.float32),
    )(p0, p1, sc)


def _dot(pooled, w_lin, b_lin):
    return pl.pallas_call(
        _dot_body,
        in_specs=[pl.BlockSpec((1, CH), lambda: (0, 0)),
                  pl.BlockSpec((CH, 1), lambda: (0, 0)),
                  pl.BlockSpec((1, 1), lambda: (0, 0))],
        out_specs=pl.BlockSpec((1, 1), lambda: (0, 0)),
        out_shape=jax.ShapeDtypeStruct((1, 1), jnp.float32),
    )(pooled, w_lin, b_lin)


def kernel(x_0, incidence_indices, W_v2e_0, W_e2v_0, W_v2e_1, W_e2v_1,
           W_lin, b_lin):
    v = incidence_indices[0]
    e = incidence_indices[1]

    def padw(idx):
        a = idx.reshape(NW, NNZ // NW)
        fill = jnp.full((NW, WNNZ - NNZ // NW), PAD_ROW, jnp.int32)
        return jnp.concatenate([a, fill], axis=1).reshape(NW, WCHUNKS, CHUNK)

    vp = padw(v)
    ep = padw(e)
    vp16 = vp.reshape(NS, ACHUNKS, CHUNK)
    ep16 = ep.reshape(NS, ACHUNKS, CHUNK)

    sv, se, snv, sn = _scales(vp16, ep16)
    sv = sv.reshape(PAD, 1)
    se = se.reshape(PAD, 1)
    snv = snv.reshape(PAD, 1)
    sn = sn.reshape(PAD, 1)

    x0p = jnp.pad(x_0, ((0, PAD - N_NODES), (0, 0)))
    t = _tbl_first(x0p, sv, W_v2e_0)
    p = _spmm(t, vp, ep)
    t = _tbl_mid(p[0], p[1], se, W_e2v_0)
    p = _spmm(t, ep, vp)
    t = _tbl_mid(p[0], p[1], snv, W_v2e_1)
    p = _spmm(t, vp, ep)
    t = _tbl_mid(p[0], p[1], se, W_e2v_1)
    p = _spmm(t, ep, vp)
    pooled = _maxpool(p[0], p[1], sn)
    out = _dot(pooled, W_lin, b_lin.reshape(1, 1))
    return out.reshape(1)


# speedup vs baseline: 4.9515x; 4.9515x over previous
import jax
import jax.numpy as jnp
from jax import lax
from jax.experimental import pallas as pl
from jax.experimental.pallas import tpu as pltpu
from jax.experimental.pallas import tpu_sc as plsc

N_NODES = 10000
N_EDGES = 10000
NNZ = 320000
CH = 128
PAD = 10240
PAD_ROW = PAD - 1
NC = 2
NS = 16
NW = NC * NS
CHUNK = 128
WCHUNKS = 80
WNNZ = WCHUNKS * CHUNK
ACHUNKS = WCHUNKS * 2
SL = PAD // NS
BLK = 256


def _rsqrt_nr(d):
    i = lax.bitcast_convert_type(d, jnp.int32)
    y = lax.bitcast_convert_type(jnp.int32(0x5F3759DF) - (i >> 1), jnp.float32)
    for _ in range(3):
        y = y * (1.5 - 0.5 * d * y * y)
    return y


def _scales_body(vp_ref, ep_ref, sv_ref, se_ref, snv_ref, sn_ref,
                 vb, eb, ones, ta, tb, vv,
                 dv_s, de_s, ne_s, nv_s, wv_s, we_s, sem):
    c = lax.axis_index("c")
    s = lax.axis_index("s")
    base = s * SL
    pltpu.sync_copy(vp_ref.at[s], vb)
    pltpu.sync_copy(ep_ref.at[s], eb)

    def z16(i, _):
        ta[pl.ds(i * 16, 16)] = jnp.zeros((16,), jnp.float32)
        return 0
    lax.fori_loop(0, SL // 16, z16, 0)

    def o16(i, _):
        ones[pl.ds(i * 16, 16)] = jnp.ones((16,), jnp.float32)
        return 0
    lax.fori_loop(0, CHUNK // 16, o16, 0)

    pltpu.sync_copy(ta, dv_s.at[pl.ds(base, SL)])
    pltpu.sync_copy(ta, de_s.at[pl.ds(base, SL)])
    pltpu.sync_copy(ta, ne_s.at[pl.ds(base, SL)])
    pltpu.sync_copy(ta, nv_s.at[pl.ds(base, SL)])
    plsc.subcore_barrier()

    def deg(i, _):
        pltpu.sync_copy(ones, dv_s.at[vb.at[i]], add=True)
        pltpu.sync_copy(ones, de_s.at[eb.at[i]], add=True)
        return 0
    lax.fori_loop(0, ACHUNKS, deg, 0)
    plsc.subcore_barrier()

    pltpu.sync_copy(dv_s.at[pl.ds(base, SL)], ta)
    pltpu.sync_copy(de_s.at[pl.ds(base, SL)], tb)

    def wvl(i, _):
        dv = jnp.maximum(ta[pl.ds(i * 16, 16)], 1.0)
        ta[pl.ds(i * 16, 16)] = _rsqrt_nr(dv)
        de = jnp.maximum(tb[pl.ds(i * 16, 16)], 1.0)
        r = _rsqrt_nr(de)
        tb[pl.ds(i * 16, 16)] = r * r * r
        return 0
    lax.fori_loop(0, SL // 16, wvl, 0)

    pltpu.sync_copy(ta, wv_s.at[pl.ds(base, SL)])
    pltpu.sync_copy(tb, we_s.at[pl.ds(base, SL)])

    @pl.when(c == 0)
    def _():
        pltpu.sync_copy(ta, sv_ref.at[pl.ds(base, SL)])
    plsc.subcore_barrier()

    def nrm(i, _):
        pltpu.async_copy(wv_s.at[vb.at[i]], vv, sem).wait()
        pltpu.sync_copy(vv, ne_s.at[eb.at[i]], add=True)
        pltpu.async_copy(we_s.at[eb.at[i]], vv, sem).wait()
        pltpu.sync_copy(vv, nv_s.at[vb.at[i]], add=True)
        return 0
    lax.fori_loop(0, ACHUNKS, nrm, 0)
    plsc.subcore_barrier()

    pltpu.sync_copy(ne_s.at[pl.ds(base, SL)], ta)

    def sel(i, _):
        ne = jnp.maximum(ta[pl.ds(i * 16, 16)], 1e-6)
        ta[pl.ds(i * 16, 16)] = tb[pl.ds(i * 16, 16)] / ne
        return 0
    lax.fori_loop(0, SL // 16, sel, 0)

    @pl.when(c == 0)
    def _():
        pltpu.sync_copy(ta, se_ref.at[pl.ds(base, SL)])

    pltpu.sync_copy(nv_s.at[pl.ds(base, SL)], ta)
    pltpu.sync_copy(wv_s.at[pl.ds(base, SL)], tb)

    def snl(i, _):
        sn = 1.0 / jnp.maximum(ta[pl.ds(i * 16, 16)], 1e-6)
        ta[pl.ds(i * 16, 16)] = sn
        tb[pl.ds(i * 16, 16)] = sn * tb[pl.ds(i * 16, 16)]
        return 0
    lax.fori_loop(0, SL // 16, snl, 0)

    @pl.when(c == 0)
    def _():
        pltpu.sync_copy(ta, sn_ref.at[pl.ds(base, SL)])
        pltpu.sync_copy(tb, snv_ref.at[pl.ds(base, SL)])


def _scales(vp16, ep16):
    mesh = plsc.VectorSubcoreMesh(core_axis_name="c", subcore_axis_name="s",
                                  num_cores=NC)
    f = pl.kernel(
        _scales_body,
        out_type=[jax.ShapeDtypeStruct((PAD,), jnp.float32)] * 4,
        mesh=mesh,
        scratch_types=[
            pltpu.VMEM((ACHUNKS, CHUNK), jnp.int32),
            pltpu.VMEM((ACHUNKS, CHUNK), jnp.int32),
            pltpu.VMEM((CHUNK,), jnp.float32),
            pltpu.VMEM((SL,), jnp.float32),
            pltpu.VMEM((SL,), jnp.float32),
            pltpu.VMEM((CHUNK,), jnp.float32),
            pltpu.VMEM_SHARED((PAD,), jnp.float32),
            pltpu.VMEM_SHARED((PAD,), jnp.float32),
            pltpu.VMEM_SHARED((PAD,), jnp.float32),
            pltpu.VMEM_SHARED((PAD,), jnp.float32),
            pltpu.VMEM_SHARED((PAD,), jnp.float32),
            pltpu.VMEM_SHARED((PAD,), jnp.float32),
            pltpu.SemaphoreType.DMA,
        ],
    )
    return f(vp16, ep16)


IPH = WCHUNKS // 2


def _spmm_body(table_ref, gi_ref, si_ref, out_ref,
               vb, eb, r0, r1, acc, sem0, sem1):
    c = lax.axis_index("c")
    s = lax.axis_index("s")
    wid = s * NC + c

    def zr(rr, _):
        def zc(j, __):
            r0[rr, pl.ds(j * 16, 16)] = jnp.zeros((16,), jnp.float32)
            return 0
        lax.fori_loop(0, CH // 16, zc, 0)
        return 0
    lax.fori_loop(0, CHUNK, zr, 0)

    def za(k, _):
        pltpu.sync_copy(r0, acc.at[pl.ds(s * SL + k * CHUNK, CHUNK)])
        return 0
    lax.fori_loop(0, SL // CHUNK, za, 0)
    plsc.subcore_barrier()

    def mo(i, _):
        d0 = pltpu.async_copy(table_ref.at[vb.at[2 * i]], r0, sem0)
        d1 = pltpu.async_copy(table_ref.at[vb.at[2 * i + 1]], r1, sem1)
        d0.wait()
        pltpu.sync_copy(r0, acc.at[eb.at[2 * i]], add=True)
        d1.wait()
        pltpu.sync_copy(r1, acc.at[eb.at[2 * i + 1]], add=True)
        return 0

    for ph in range(WCHUNKS // IPH):
        pltpu.sync_copy(gi_ref.at[wid, pl.ds(ph * IPH, IPH)], vb)
        pltpu.sync_copy(si_ref.at[wid, pl.ds(ph * IPH, IPH)], eb)
        lax.fori_loop(0, IPH // 2, mo, 0)
    plsc.subcore_barrier()

    def wb(k, _):
        row = s * SL + k * CHUNK
        pltpu.sync_copy(acc.at[pl.ds(row, CHUNK)], r0)
        pltpu.sync_copy(r0, out_ref.at[c, pl.ds(row, CHUNK)])
        return 0
    lax.fori_loop(0, SL // CHUNK, wb, 0)


def _spmm(table, gidx, sidx):
    mesh = plsc.VectorSubcoreMesh(core_axis_name="c", subcore_axis_name="s",
                                  num_cores=NC)
    f = pl.kernel(
        _spmm_body,
        out_type=jax.ShapeDtypeStruct((NC, PAD, CH), jnp.float32),
        mesh=mesh,
        scratch_types=[
            pltpu.VMEM((IPH, CHUNK), jnp.int32),
            pltpu.VMEM((IPH, CHUNK), jnp.int32),
            pltpu.VMEM((CHUNK, CH), jnp.float32),
            pltpu.VMEM((CHUNK, CH), jnp.float32),
            pltpu.VMEM_SHARED((PAD, CH), jnp.float32),
            pltpu.SemaphoreType.DMA,
            pltpu.SemaphoreType.DMA,
        ],
    )
    return f(table, gidx, sidx)


def _tbl_first_body(x_ref, s_ref, w_ref, o_ref):
    o_ref[...] = lax.dot_general(x_ref[...] * s_ref[...], w_ref[...],
                                 (((1,), (0,)), ((), ())),
                                 preferred_element_type=jnp.float32)


def _tbl_mid_body(p0_ref, p1_ref, s_ref, w_ref, o_ref):
    h = jnp.maximum(p0_ref[...] + p1_ref[...], 0.0) * s_ref[...]
    o_ref[...] = lax.dot_general(h, w_ref[...], (((1,), (0,)), ((), ())),
                                 preferred_element_type=jnp.float32)


def _max_body(p0_ref, p1_ref, s_ref, o_ref):
    i = pl.program_id(0)
    h = jnp.maximum(p0_ref[...] + p1_ref[...], 0.0) * s_ref[...]
    m = jnp.max(h, axis=0, keepdims=True)

    @pl.when(i == 0)
    def _():
        o_ref[...] = m

    @pl.when(i != 0)
    def _():
        o_ref[...] = jnp.maximum(o_ref[...], m)


def _dot_body(p_ref, w_ref, b_ref, o_ref):
    o_ref[...] = lax.dot_general(p_ref[...], w_ref[...],
                                 (((1,), (0,)), ((), ())),
                                 preferred_element_type=jnp.float32) + b_ref[...]


def _tbl_first(x, sv, w):
    return pl.pallas_call(
        _tbl_first_body,
        grid=(PAD // BLK,),
        in_specs=[pl.BlockSpec((BLK, CH), lambda i: (i, 0)),
                  pl.BlockSpec((BLK, 1), lambda i: (i, 0)),
                  pl.BlockSpec((CH, CH), lambda i: (0, 0))],
        out_specs=pl.BlockSpec((BLK, CH), lambda i: (i, 0)),
        out_shape=jax.ShapeDtypeStruct((PAD, CH), jnp.float32),
    )(x, sv, w)


def _tbl_mid(p0, p1, sc, w):
    return pl.pallas_call(
        _tbl_mid_body,
        grid=(PAD // BLK,),
        in_specs=[pl.BlockSpec((BLK, CH), lambda i: (i, 0)),
                  pl.BlockSpec((BLK, CH), lambda i: (i, 0)),
                  pl.BlockSpec((BLK, 1), lambda i: (i, 0)),
                  pl.BlockSpec((CH, CH), lambda i: (0, 0))],
        out_specs=pl.BlockSpec((BLK, CH), lambda i: (i, 0)),
        out_shape=jax.ShapeDtypeStruct((PAD, CH), jnp.float32),
    )(p0, p1, sc, w)


def _maxpool(p0, p1, sc):
    return pl.pallas_call(
        _max_body,
        grid=(PAD // BLK,),
        in_specs=[pl.BlockSpec((BLK, CH), lambda i: (i, 0)),
                  pl.BlockSpec((BLK, CH), lambda i: (i, 0)),
                  pl.BlockSpec((BLK, 1), lambda i: (i, 0))],
        out_specs=pl.BlockSpec((1, CH), lambda i: (0, 0)),
        out_shape=jax.ShapeDtypeStruct((1, CH), jnp.float32),
    )(p0, p1, sc)


def _dot(pooled, w_lin, b_lin):
    return pl.pallas_call(
        _dot_body,
        in_specs=[pl.BlockSpec((1, CH), lambda: (0, 0)),
                  pl.BlockSpec((CH, 1), lambda: (0, 0)),
                  pl.BlockSpec((1, 1), lambda: (0, 0))],
        out_specs=pl.BlockSpec((1, 1), lambda: (0, 0)),
        out_shape=jax.ShapeDtypeStruct((1, 1), jnp.float32),
    )(pooled, w_lin, b_lin)


def kernel(x_0, incidence_indices, W_v2e_0, W_e2v_0, W_v2e_1, W_e2v_1,
           W_lin, b_lin):
    v = incidence_indices[0]
    e = incidence_indices[1]

    def padw(idx):
        a = idx.reshape(NW, NNZ // NW)
        fill = jnp.full((NW, WNNZ - NNZ // NW), PAD_ROW, jnp.int32)
        return jnp.concatenate([a, fill], axis=1).reshape(NW, WCHUNKS, CHUNK)

    vp = padw(v)
    ep = padw(e)
    vp16 = vp.reshape(NS, ACHUNKS, CHUNK)
    ep16 = ep.reshape(NS, ACHUNKS, CHUNK)

    sv, se, snv, sn = _scales(vp16, ep16)
    sv = sv.reshape(PAD, 1)
    se = se.reshape(PAD, 1)
    snv = snv.reshape(PAD, 1)
    sn = sn.reshape(PAD, 1)

    x0p = jnp.pad(x_0, ((0, PAD - N_NODES), (0, 0)))
    t = _tbl_first(x0p, sv, W_v2e_0)
    p = _spmm(t, vp, ep)
    t = _tbl_mid(p[0], p[1], se, W_e2v_0)
    p = _spmm(t, ep, vp)
    t = _tbl_mid(p[0], p[1], snv, W_v2e_1)
    p = _spmm(t, vp, ep)
    t = _tbl_mid(p[0], p[1], se, W_e2v_1)
    p = _spmm(t, ep, vp)
    pooled = _maxpool(p[0], p[1], sn)
    out = _dot(pooled, W_lin, b_lin.reshape(1, 1))
    return out.reshape(1)


# speedup vs baseline: 5.3708x; 1.0847x over previous
import jax
import jax.numpy as jnp
from jax import lax
from jax.experimental import pallas as pl
from jax.experimental.pallas import tpu as pltpu
from jax.experimental.pallas import tpu_sc as plsc

N_NODES = 10000
N_EDGES = 10000
NNZ = 320000
CH = 128
PAD = 10240
PAD_ROW = PAD - 1
NC = 2
NS = 16
NW = NC * NS
CHUNK = 128
WCHUNKS = 80
WNNZ = WCHUNKS * CHUNK
ACHUNKS = WCHUNKS * 2
SL = PAD // NS
BLK = 256


def _rsqrt_nr(d):
    i = lax.bitcast_convert_type(d, jnp.int32)
    y = lax.bitcast_convert_type(jnp.int32(0x5F3759DF) - (i >> 1), jnp.float32)
    for _ in range(3):
        y = y * (1.5 - 0.5 * d * y * y)
    return y


def _scales_body(vp_ref, ep_ref, sv_ref, se_ref, snv_ref, sn_ref,
                 vb, eb, ones, ta, tb, vv,
                 dv_s, de_s, ne_s, nv_s, wv_s, we_s, sem):
    c = lax.axis_index("c")
    s = lax.axis_index("s")
    base = s * SL
    pltpu.sync_copy(vp_ref.at[s], vb)
    pltpu.sync_copy(ep_ref.at[s], eb)

    def z16(i, _):
        ta[pl.ds(i * 16, 16)] = jnp.zeros((16,), jnp.float32)
        return 0
    lax.fori_loop(0, SL // 16, z16, 0)

    def o16(i, _):
        ones[pl.ds(i * 16, 16)] = jnp.ones((16,), jnp.float32)
        return 0
    lax.fori_loop(0, CHUNK // 16, o16, 0)

    pltpu.sync_copy(ta, dv_s.at[pl.ds(base, SL)])
    pltpu.sync_copy(ta, de_s.at[pl.ds(base, SL)])
    pltpu.sync_copy(ta, ne_s.at[pl.ds(base, SL)])
    pltpu.sync_copy(ta, nv_s.at[pl.ds(base, SL)])
    plsc.subcore_barrier()

    def deg(i, _):
        pltpu.sync_copy(ones, dv_s.at[vb.at[i]], add=True)
        pltpu.sync_copy(ones, de_s.at[eb.at[i]], add=True)
        return 0
    lax.fori_loop(0, ACHUNKS, deg, 0)
    plsc.subcore_barrier()

    pltpu.sync_copy(dv_s.at[pl.ds(base, SL)], ta)
    pltpu.sync_copy(de_s.at[pl.ds(base, SL)], tb)

    def wvl(i, _):
        dv = jnp.maximum(ta[pl.ds(i * 16, 16)], 1.0)
        ta[pl.ds(i * 16, 16)] = _rsqrt_nr(dv)
        de = jnp.maximum(tb[pl.ds(i * 16, 16)], 1.0)
        r = _rsqrt_nr(de)
        tb[pl.ds(i * 16, 16)] = r * r * r
        return 0
    lax.fori_loop(0, SL // 16, wvl, 0)

    pltpu.sync_copy(ta, wv_s.at[pl.ds(base, SL)])
    pltpu.sync_copy(tb, we_s.at[pl.ds(base, SL)])

    @pl.when(c == 0)
    def _():
        pltpu.sync_copy(ta, sv_ref.at[pl.ds(base, SL)])
    plsc.subcore_barrier()

    def nrm(i, _):
        pltpu.async_copy(wv_s.at[vb.at[i]], vv, sem).wait()
        pltpu.sync_copy(vv, ne_s.at[eb.at[i]], add=True)
        pltpu.async_copy(we_s.at[eb.at[i]], vv, sem).wait()
        pltpu.sync_copy(vv, nv_s.at[vb.at[i]], add=True)
        return 0
    lax.fori_loop(0, ACHUNKS, nrm, 0)
    plsc.subcore_barrier()

    pltpu.sync_copy(ne_s.at[pl.ds(base, SL)], ta)

    def sel(i, _):
        ne = jnp.maximum(ta[pl.ds(i * 16, 16)], 1e-6)
        ta[pl.ds(i * 16, 16)] = tb[pl.ds(i * 16, 16)] / ne
        return 0
    lax.fori_loop(0, SL // 16, sel, 0)

    @pl.when(c == 0)
    def _():
        pltpu.sync_copy(ta, se_ref.at[pl.ds(base, SL)])

    pltpu.sync_copy(nv_s.at[pl.ds(base, SL)], ta)
    pltpu.sync_copy(wv_s.at[pl.ds(base, SL)], tb)

    def snl(i, _):
        sn = 1.0 / jnp.maximum(ta[pl.ds(i * 16, 16)], 1e-6)
        ta[pl.ds(i * 16, 16)] = sn
        tb[pl.ds(i * 16, 16)] = sn * tb[pl.ds(i * 16, 16)]
        return 0
    lax.fori_loop(0, SL // 16, snl, 0)

    @pl.when(c == 0)
    def _():
        pltpu.sync_copy(ta, sn_ref.at[pl.ds(base, SL)])
        pltpu.sync_copy(tb, snv_ref.at[pl.ds(base, SL)])


def _scales(vp16, ep16):
    mesh = plsc.VectorSubcoreMesh(core_axis_name="c", subcore_axis_name="s",
                                  num_cores=NC)
    f = pl.kernel(
        _scales_body,
        out_type=[jax.ShapeDtypeStruct((PAD,), jnp.float32)] * 4,
        mesh=mesh,
        scratch_types=[
            pltpu.VMEM((ACHUNKS, CHUNK), jnp.int32),
            pltpu.VMEM((ACHUNKS, CHUNK), jnp.int32),
            pltpu.VMEM((CHUNK,), jnp.float32),
            pltpu.VMEM((SL,), jnp.float32),
            pltpu.VMEM((SL,), jnp.float32),
            pltpu.VMEM((CHUNK,), jnp.float32),
            pltpu.VMEM_SHARED((PAD,), jnp.float32),
            pltpu.VMEM_SHARED((PAD,), jnp.float32),
            pltpu.VMEM_SHARED((PAD,), jnp.float32),
            pltpu.VMEM_SHARED((PAD,), jnp.float32),
            pltpu.VMEM_SHARED((PAD,), jnp.float32),
            pltpu.VMEM_SHARED((PAD,), jnp.float32),
            pltpu.SemaphoreType.DMA,
        ],
    )
    return f(vp16, ep16)


IPH = WCHUNKS // 2


def _spmm_body(table_ref, gi_ref, si_ref, out_ref,
               vb, eb, r0, r1, acc, sem0, sem1):
    c = lax.axis_index("c")
    s = lax.axis_index("s")
    wid = s * NC + c

    def zr(rr, _):
        def zc(j, __):
            r0[rr, pl.ds(j * 16, 16)] = jnp.zeros((16,), jnp.float32)
            return 0
        lax.fori_loop(0, CH // 16, zc, 0)
        return 0
    lax.fori_loop(0, CHUNK, zr, 0)

    def za(k, _):
        pltpu.sync_copy(r0, acc.at[pl.ds(s * SL + k * CHUNK, CHUNK)])
        return 0
    lax.fori_loop(0, SL // CHUNK, za, 0)
    plsc.subcore_barrier()

    def mo(i, _):
        pltpu.make_async_copy(table_ref.at[vb.at[0]], r0, sem0).wait()
        pltpu.sync_copy(r0, acc.at[eb.at[2 * i]], add=True)
        pltpu.async_copy(table_ref.at[vb.at[2 * i + 2]], r0, sem0)
        pltpu.make_async_copy(table_ref.at[vb.at[0]], r1, sem1).wait()
        pltpu.sync_copy(r1, acc.at[eb.at[2 * i + 1]], add=True)
        pltpu.async_copy(table_ref.at[vb.at[2 * i + 3]], r1, sem1)
        return 0

    for ph in range(WCHUNKS // IPH):
        pltpu.sync_copy(gi_ref.at[wid, pl.ds(ph * IPH, IPH)], vb)
        pltpu.sync_copy(si_ref.at[wid, pl.ds(ph * IPH, IPH)], eb)
        pltpu.async_copy(table_ref.at[vb.at[0]], r0, sem0)
        pltpu.async_copy(table_ref.at[vb.at[1]], r1, sem1)
        lax.fori_loop(0, IPH // 2 - 1, mo, 0)
        pltpu.make_async_copy(table_ref.at[vb.at[0]], r0, sem0).wait()
        pltpu.sync_copy(r0, acc.at[eb.at[IPH - 2]], add=True)
        pltpu.make_async_copy(table_ref.at[vb.at[0]], r1, sem1).wait()
        pltpu.sync_copy(r1, acc.at[eb.at[IPH - 1]], add=True)
    plsc.subcore_barrier()

    def wb(k, _):
        row = s * SL + k * CHUNK
        pltpu.sync_copy(acc.at[pl.ds(row, CHUNK)], r0)
        pltpu.sync_copy(r0, out_ref.at[c, pl.ds(row, CHUNK)])
        return 0
    lax.fori_loop(0, SL // CHUNK, wb, 0)


def _spmm(table, gidx, sidx):
    mesh = plsc.VectorSubcoreMesh(core_axis_name="c", subcore_axis_name="s",
                                  num_cores=NC)
    f = pl.kernel(
        _spmm_body,
        out_type=jax.ShapeDtypeStruct((NC, PAD, CH), jnp.float32),
        mesh=mesh,
        scratch_types=[
            pltpu.VMEM((IPH, CHUNK), jnp.int32),
            pltpu.VMEM((IPH, CHUNK), jnp.int32),
            pltpu.VMEM((CHUNK, CH), jnp.float32),
            pltpu.VMEM((CHUNK, CH), jnp.float32),
            pltpu.VMEM_SHARED((PAD, CH), jnp.float32),
            pltpu.SemaphoreType.DMA,
            pltpu.SemaphoreType.DMA,
        ],
    )
    return f(table, gidx, sidx)


def _tbl_first_body(x_ref, s_ref, w_ref, o_ref):
    o_ref[...] = lax.dot_general(x_ref[...] * s_ref[...], w_ref[...],
                                 (((1,), (0,)), ((), ())),
                                 preferred_element_type=jnp.float32)


def _tbl_mid_body(p0_ref, p1_ref, s_ref, w_ref, o_ref):
    h = jnp.maximum(p0_ref[...] + p1_ref[...], 0.0) * s_ref[...]
    o_ref[...] = lax.dot_general(h, w_ref[...], (((1,), (0,)), ((), ())),
                                 preferred_element_type=jnp.float32)


def _max_body(p0_ref, p1_ref, s_ref, o_ref):
    i = pl.program_id(0)
    h = jnp.maximum(p0_ref[...] + p1_ref[...], 0.0) * s_ref[...]
    m = jnp.max(h, axis=0, keepdims=True)

    @pl.when(i == 0)
    def _():
        o_ref[...] = m

    @pl.when(i != 0)
    def _():
        o_ref[...] = jnp.maximum(o_ref[...], m)


def _dot_body(p_ref, w_ref, b_ref, o_ref):
    o_ref[...] = lax.dot_general(p_ref[...], w_ref[...],
                                 (((1,), (0,)), ((), ())),
                                 preferred_element_type=jnp.float32) + b_ref[...]


def _tbl_first(x, sv, w):
    return pl.pallas_call(
        _tbl_first_body,
        grid=(PAD // BLK,),
        in_specs=[pl.BlockSpec((BLK, CH), lambda i: (i, 0)),
                  pl.BlockSpec((BLK, 1), lambda i: (i, 0)),
                  pl.BlockSpec((CH, CH), lambda i: (0, 0))],
        out_specs=pl.BlockSpec((BLK, CH), lambda i: (i, 0)),
        out_shape=jax.ShapeDtypeStruct((PAD, CH), jnp.float32),
    )(x, sv, w)


def _tbl_mid(p0, p1, sc, w):
    return pl.pallas_call(
        _tbl_mid_body,
        grid=(PAD // BLK,),
        in_specs=[pl.BlockSpec((BLK, CH), lambda i: (i, 0)),
                  pl.BlockSpec((BLK, CH), lambda i: (i, 0)),
                  pl.BlockSpec((BLK, 1), lambda i: (i, 0)),
                  pl.BlockSpec((CH, CH), lambda i: (0, 0))],
        out_specs=pl.BlockSpec((BLK, CH), lambda i: (i, 0)),
        out_shape=jax.ShapeDtypeStruct((PAD, CH), jnp.float32),
    )(p0, p1, sc, w)


def _maxpool(p0, p1, sc):
    return pl.pallas_call(
        _max_body,
        grid=(PAD // BLK,),
        in_specs=[pl.BlockSpec((BLK, CH), lambda i: (i, 0)),
                  pl.BlockSpec((BLK, CH), lambda i: (i, 0)),
                  pl.BlockSpec((BLK, 1), lambda i: (i, 0))],
        out_specs=pl.BlockSpec((1, CH), lambda i: (0, 0)),
        out_shape=jax.ShapeDtypeStruct((1, CH), jnp.float32),
    )(p0, p1, sc)


def _dot(pooled, w_lin, b_lin):
    return pl.pallas_call(
        _dot_body,
        in_specs=[pl.BlockSpec((1, CH), lambda: (0, 0)),
                  pl.BlockSpec((CH, 1), lambda: (0, 0)),
                  pl.BlockSpec((1, 1), lambda: (0, 0))],
        out_specs=pl.BlockSpec((1, 1), lambda: (0, 0)),
        out_shape=jax.ShapeDtypeStruct((1, 1), jnp.float32),
    )(pooled, w_lin, b_lin)


def kernel(x_0, incidence_indices, W_v2e_0, W_e2v_0, W_v2e_1, W_e2v_1,
           W_lin, b_lin):
    v = incidence_indices[0]
    e = incidence_indices[1]

    def padw(idx):
        a = idx.reshape(NW, NNZ // NW)
        fill = jnp.full((NW, WNNZ - NNZ // NW), PAD_ROW, jnp.int32)
        return jnp.concatenate([a, fill], axis=1).reshape(NW, WCHUNKS, CHUNK)

    vp = padw(v)
    ep = padw(e)
    vp16 = vp.reshape(NS, ACHUNKS, CHUNK)
    ep16 = ep.reshape(NS, ACHUNKS, CHUNK)

    sv, se, snv, sn = _scales(vp16, ep16)
    sv = sv.reshape(PAD, 1)
    se = se.reshape(PAD, 1)
    snv = snv.reshape(PAD, 1)
    sn = sn.reshape(PAD, 1)

    x0p = jnp.pad(x_0, ((0, PAD - N_NODES), (0, 0)))
    t = _tbl_first(x0p, sv, W_v2e_0)
    p = _spmm(t, vp, ep)
    t = _tbl_mid(p[0], p[1], se, W_e2v_0)
    p = _spmm(t, ep, vp)
    t = _tbl_mid(p[0], p[1], snv, W_v2e_1)
    p = _spmm(t, vp, ep)
    t = _tbl_mid(p[0], p[1], se, W_e2v_1)
    p = _spmm(t, ep, vp)
    pooled = _maxpool(p[0], p[1], sn)
    out = _dot(pooled, W_lin, b_lin.reshape(1, 1))
    return out.reshape(1)


# speedup vs baseline: 5.4681x; 1.0181x over previous
import jax
import jax.numpy as jnp
from jax import lax
from jax.experimental import pallas as pl
from jax.experimental.pallas import tpu as pltpu
from jax.experimental.pallas import tpu_sc as plsc

N_NODES = 10000
N_EDGES = 10000
NNZ = 320000
CH = 128
PAD = 10240
PAD_ROW = PAD - 1
NC = 2
NS = 16
NW = NC * NS
CHUNK = 128
WCHUNKS = 80
WNNZ = WCHUNKS * CHUNK
ACHUNKS = WCHUNKS * 2
SL = PAD // NS
BLK = 256


def _rsqrt_nr(d):
    i = lax.bitcast_convert_type(d, jnp.int32)
    y = lax.bitcast_convert_type(jnp.int32(0x5F3759DF) - (i >> 1), jnp.float32)
    for _ in range(3):
        y = y * (1.5 - 0.5 * d * y * y)
    return y


def _scales_body(vp_ref, ep_ref, sv_ref, se_ref, snv_ref, sn_ref,
                 vb, eb, ones, ta, tb, vv,
                 dv_s, de_s, ne_s, nv_s, wv_s, we_s, sem):
    c = lax.axis_index("c")
    s = lax.axis_index("s")
    base = s * SL
    pltpu.sync_copy(vp_ref.at[s], vb)
    pltpu.sync_copy(ep_ref.at[s], eb)

    def z16(i, _):
        ta[pl.ds(i * 16, 16)] = jnp.zeros((16,), jnp.float32)
        return 0
    lax.fori_loop(0, SL // 16, z16, 0)

    def o16(i, _):
        ones[pl.ds(i * 16, 16)] = jnp.ones((16,), jnp.float32)
        return 0
    lax.fori_loop(0, CHUNK // 16, o16, 0)

    pltpu.sync_copy(ta, dv_s.at[pl.ds(base, SL)])
    pltpu.sync_copy(ta, de_s.at[pl.ds(base, SL)])
    pltpu.sync_copy(ta, ne_s.at[pl.ds(base, SL)])
    pltpu.sync_copy(ta, nv_s.at[pl.ds(base, SL)])
    plsc.subcore_barrier()

    def deg(i, _):
        pltpu.sync_copy(ones, dv_s.at[vb.at[i]], add=True)
        pltpu.sync_copy(ones, de_s.at[eb.at[i]], add=True)
        return 0
    lax.fori_loop(0, ACHUNKS, deg, 0)
    plsc.subcore_barrier()

    pltpu.sync_copy(dv_s.at[pl.ds(base, SL)], ta)
    pltpu.sync_copy(de_s.at[pl.ds(base, SL)], tb)

    def wvl(i, _):
        dv = jnp.maximum(ta[pl.ds(i * 16, 16)], 1.0)
        ta[pl.ds(i * 16, 16)] = _rsqrt_nr(dv)
        de = jnp.maximum(tb[pl.ds(i * 16, 16)], 1.0)
        r = _rsqrt_nr(de)
        tb[pl.ds(i * 16, 16)] = r * r * r
        return 0
    lax.fori_loop(0, SL // 16, wvl, 0)

    pltpu.sync_copy(ta, wv_s.at[pl.ds(base, SL)])
    pltpu.sync_copy(tb, we_s.at[pl.ds(base, SL)])

    @pl.when(c == 0)
    def _():
        pltpu.sync_copy(ta, sv_ref.at[pl.ds(base, SL)])
    plsc.subcore_barrier()

    def nrm(i, _):
        pltpu.async_copy(wv_s.at[vb.at[i]], vv, sem).wait()
        pltpu.sync_copy(vv, ne_s.at[eb.at[i]], add=True)
        pltpu.async_copy(we_s.at[eb.at[i]], vv, sem).wait()
        pltpu.sync_copy(vv, nv_s.at[vb.at[i]], add=True)
        return 0
    lax.fori_loop(0, ACHUNKS, nrm, 0)
    plsc.subcore_barrier()

    pltpu.sync_copy(ne_s.at[pl.ds(base, SL)], ta)

    def sel(i, _):
        ne = jnp.maximum(ta[pl.ds(i * 16, 16)], 1e-6)
        ta[pl.ds(i * 16, 16)] = tb[pl.ds(i * 16, 16)] / ne
        return 0
    lax.fori_loop(0, SL // 16, sel, 0)

    @pl.when(c == 0)
    def _():
        pltpu.sync_copy(ta, se_ref.at[pl.ds(base, SL)])

    pltpu.sync_copy(nv_s.at[pl.ds(base, SL)], ta)
    pltpu.sync_copy(wv_s.at[pl.ds(base, SL)], tb)

    def snl(i, _):
        sn = 1.0 / jnp.maximum(ta[pl.ds(i * 16, 16)], 1e-6)
        ta[pl.ds(i * 16, 16)] = sn
        tb[pl.ds(i * 16, 16)] = sn * tb[pl.ds(i * 16, 16)]
        return 0
    lax.fori_loop(0, SL // 16, snl, 0)

    @pl.when(c == 0)
    def _():
        pltpu.sync_copy(ta, sn_ref.at[pl.ds(base, SL)])
        pltpu.sync_copy(tb, snv_ref.at[pl.ds(base, SL)])


def _scales(vp16, ep16):
    mesh = plsc.VectorSubcoreMesh(core_axis_name="c", subcore_axis_name="s",
                                  num_cores=NC)
    f = pl.kernel(
        _scales_body,
        out_type=[jax.ShapeDtypeStruct((PAD,), jnp.float32)] * 4,
        mesh=mesh,
        scratch_types=[
            pltpu.VMEM((ACHUNKS, CHUNK), jnp.int32),
            pltpu.VMEM((ACHUNKS, CHUNK), jnp.int32),
            pltpu.VMEM((CHUNK,), jnp.float32),
            pltpu.VMEM((SL,), jnp.float32),
            pltpu.VMEM((SL,), jnp.float32),
            pltpu.VMEM((CHUNK,), jnp.float32),
            pltpu.VMEM_SHARED((PAD,), jnp.float32),
            pltpu.VMEM_SHARED((PAD,), jnp.float32),
            pltpu.VMEM_SHARED((PAD,), jnp.float32),
            pltpu.VMEM_SHARED((PAD,), jnp.float32),
            pltpu.VMEM_SHARED((PAD,), jnp.float32),
            pltpu.VMEM_SHARED((PAD,), jnp.float32),
            pltpu.SemaphoreType.DMA,
        ],
    )
    return f(vp16, ep16)


IPH = WCHUNKS // 2


NB = 4
HCH = CHUNK // 2


def _spmm_body(table_ref, gi_ref, si_ref, out_ref,
               vb, eb, r0, r1, r2, r3, acc, sem0, sem1, sem2, sem3):
    c = lax.axis_index("c")
    s = lax.axis_index("s")
    wid = s * NC + c
    rbufs = (r0, r1, r2, r3)
    sems = (sem0, sem1, sem2, sem3)

    def zr(rr, _):
        def zc(j, __):
            r0[rr, pl.ds(j * 16, 16)] = jnp.zeros((16,), jnp.float32)
            r1[rr, pl.ds(j * 16, 16)] = jnp.zeros((16,), jnp.float32)
            return 0
        lax.fori_loop(0, CH // 16, zc, 0)
        return 0
    lax.fori_loop(0, HCH, zr, 0)

    def za(k, _):
        base = s * SL + k * CHUNK
        pltpu.sync_copy(r0, acc.at[pl.ds(base, HCH)])
        pltpu.sync_copy(r1, acc.at[pl.ds(base + HCH, HCH)])
        return 0
    lax.fori_loop(0, SL // CHUNK, za, 0)
    plsc.subcore_barrier()

    def gidx(i, b):
        return vb.at[2 * i + b // 2, pl.ds((b % 2) * HCH, HCH)]

    def sidx(i, b):
        return eb.at[2 * i + b // 2, pl.ds((b % 2) * HCH, HCH)]

    def mo(i, _):
        for b in range(NB):
            pltpu.make_async_copy(table_ref.at[gidx(0, b)], rbufs[b],
                                  sems[b]).wait()
            pltpu.sync_copy(rbufs[b], acc.at[sidx(i, b)], add=True)
            pltpu.async_copy(table_ref.at[gidx(i + 1, b)], rbufs[b], sems[b])
        return 0

    NIT = IPH // 2
    for ph in range(WCHUNKS // IPH):
        pltpu.sync_copy(gi_ref.at[wid, pl.ds(ph * IPH, IPH)], vb)
        pltpu.sync_copy(si_ref.at[wid, pl.ds(ph * IPH, IPH)], eb)
        for b in range(NB):
            pltpu.async_copy(table_ref.at[gidx(0, b)], rbufs[b], sems[b])
        lax.fori_loop(0, NIT - 1, mo, 0)
        for b in range(NB):
            pltpu.make_async_copy(table_ref.at[gidx(0, b)], rbufs[b],
                                  sems[b]).wait()
            pltpu.sync_copy(rbufs[b], acc.at[sidx(NIT - 1, b)], add=True)
    plsc.subcore_barrier()

    def wb(k, _):
        row = s * SL + k * HCH
        pltpu.sync_copy(acc.at[pl.ds(row, HCH)], r0)
        pltpu.sync_copy(r0, out_ref.at[c, pl.ds(row, HCH)])
        return 0
    lax.fori_loop(0, SL // HCH, wb, 0)


def _spmm(table, gidx, sidx):
    mesh = plsc.VectorSubcoreMesh(core_axis_name="c", subcore_axis_name="s",
                                  num_cores=NC)
    f = pl.kernel(
        _spmm_body,
        out_type=jax.ShapeDtypeStruct((NC, PAD, CH), jnp.float32),
        mesh=mesh,
        scratch_types=[
            pltpu.VMEM((IPH, CHUNK), jnp.int32),
            pltpu.VMEM((IPH, CHUNK), jnp.int32),
            pltpu.VMEM((CHUNK // 2, CH), jnp.float32),
            pltpu.VMEM((CHUNK // 2, CH), jnp.float32),
            pltpu.VMEM((CHUNK // 2, CH), jnp.float32),
            pltpu.VMEM((CHUNK // 2, CH), jnp.float32),
            pltpu.VMEM_SHARED((PAD, CH), jnp.float32),
            pltpu.SemaphoreType.DMA,
            pltpu.SemaphoreType.DMA,
            pltpu.SemaphoreType.DMA,
            pltpu.SemaphoreType.DMA,
        ],
    )
    return f(table, gidx, sidx)


def _tbl_first_body(x_ref, s_ref, w_ref, o_ref):
    o_ref[...] = lax.dot_general(x_ref[...] * s_ref[...], w_ref[...],
                                 (((1,), (0,)), ((), ())),
                                 preferred_element_type=jnp.float32)


def _tbl_mid_body(p0_ref, p1_ref, s_ref, w_ref, o_ref):
    h = jnp.maximum(p0_ref[...] + p1_ref[...], 0.0) * s_ref[...]
    o_ref[...] = lax.dot_general(h, w_ref[...], (((1,), (0,)), ((), ())),
                                 preferred_element_type=jnp.float32)


def _max_body(p0_ref, p1_ref, s_ref, o_ref):
    i = pl.program_id(0)
    h = jnp.maximum(p0_ref[...] + p1_ref[...], 0.0) * s_ref[...]
    m = jnp.max(h, axis=0, keepdims=True)

    @pl.when(i == 0)
    def _():
        o_ref[...] = m

    @pl.when(i != 0)
    def _():
        o_ref[...] = jnp.maximum(o_ref[...], m)


def _dot_body(p_ref, w_ref, b_ref, o_ref):
    o_ref[...] = lax.dot_general(p_ref[...], w_ref[...],
                                 (((1,), (0,)), ((), ())),
                                 preferred_element_type=jnp.float32) + b_ref[...]


def _tbl_first(x, sv, w):
    return pl.pallas_call(
        _tbl_first_body,
        grid=(PAD // BLK,),
        in_specs=[pl.BlockSpec((BLK, CH), lambda i: (i, 0)),
                  pl.BlockSpec((BLK, 1), lambda i: (i, 0)),
                  pl.BlockSpec((CH, CH), lambda i: (0, 0))],
        out_specs=pl.BlockSpec((BLK, CH), lambda i: (i, 0)),
        out_shape=jax.ShapeDtypeStruct((PAD, CH), jnp.float32),
    )(x, sv, w)


def _tbl_mid(p0, p1, sc, w):
    return pl.pallas_call(
        _tbl_mid_body,
        grid=(PAD // BLK,),
        in_specs=[pl.BlockSpec((BLK, CH), lambda i: (i, 0)),
                  pl.BlockSpec((BLK, CH), lambda i: (i, 0)),
                  pl.BlockSpec((BLK, 1), lambda i: (i, 0)),
                  pl.BlockSpec((CH, CH), lambda i: (0, 0))],
        out_specs=pl.BlockSpec((BLK, CH), lambda i: (i, 0)),
        out_shape=jax.ShapeDtypeStruct((PAD, CH), jnp.float32),
    )(p0, p1, sc, w)


def _maxpool(p0, p1, sc):
    return pl.pallas_call(
        _max_body,
        grid=(PAD // BLK,),
        in_specs=[pl.BlockSpec((BLK, CH), lambda i: (i, 0)),
                  pl.BlockSpec((BLK, CH), lambda i: (i, 0)),
                  pl.BlockSpec((BLK, 1), lambda i: (i, 0))],
        out_specs=pl.BlockSpec((1, CH), lambda i: (0, 0)),
        out_shape=jax.ShapeDtypeStruct((1, CH), jnp.float32),
    )(p0, p1, sc)


def _dot(pooled, w_lin, b_lin):
    return pl.pallas_call(
        _dot_body,
        in_specs=[pl.BlockSpec((1, CH), lambda: (0, 0)),
                  pl.BlockSpec((CH, 1), lambda: (0, 0)),
                  pl.BlockSpec((1, 1), lambda: (0, 0))],
        out_specs=pl.BlockSpec((1, 1), lambda: (0, 0)),
        out_shape=jax.ShapeDtypeStruct((1, 1), jnp.float32),
    )(pooled, w_lin, b_lin)


def kernel(x_0, incidence_indices, W_v2e_0, W_e2v_0, W_v2e_1, W_e2v_1,
           W_lin, b_lin):
    v = incidence_indices[0]
    e = incidence_indices[1]

    def padw(idx):
        a = idx.reshape(NW, NNZ // NW)
        fill = jnp.full((NW, WNNZ - NNZ // NW), PAD_ROW, jnp.int32)
        return jnp.concatenate([a, fill], axis=1).reshape(NW, WCHUNKS, CHUNK)

    vp = padw(v)
    ep = padw(e)
    vp16 = vp.reshape(NS, ACHUNKS, CHUNK)
    ep16 = ep.reshape(NS, ACHUNKS, CHUNK)

    sv, se, snv, sn = _scales(vp16, ep16)
    sv = sv.reshape(PAD, 1)
    se = se.reshape(PAD, 1)
    snv = snv.reshape(PAD, 1)
    sn = sn.reshape(PAD, 1)

    x0p = jnp.pad(x_0, ((0, PAD - N_NODES), (0, 0)))
    t = _tbl_first(x0p, sv, W_v2e_0)
    p = _spmm(t, vp, ep)
    t = _tbl_mid(p[0], p[1], se, W_e2v_0)
    p = _spmm(t, ep, vp)
    t = _tbl_mid(p[0], p[1], snv, W_v2e_1)
    p = _spmm(t, vp, ep)
    t = _tbl_mid(p[0], p[1], se, W_e2v_1)
    p = _spmm(t, ep, vp)
    pooled = _maxpool(p[0], p[1], sn)
    out = _dot(pooled, W_lin, b_lin.reshape(1, 1))
    return out.reshape(1)
